# Initial kernel scaffold; baseline (speedup 1.0000x reference)
import jax
import jax.numpy as jnp
from jax.experimental import pallas as pl


def kernel(features, edge_index, edge_attr, params):
    raise NotImplementedError("write your pallas kernel here")


# speedup vs baseline: 1.3513x; 1.3513x over previous
import functools

import jax
import jax.numpy as jnp
from jax import lax
from jax.experimental import pallas as pl
from jax.experimental.pallas import tpu as pltpu
from jax.experimental.pallas import tpu_sc as plsc

N = 16200
E = N * 8
F = 42
D = 128
DEC = 64
NB = 3
NM = 3

TRN = 1800
NRT = N // TRN
TRE = 1600
ERT = E // TRE

CH = 128
NCHUNK = (E + CH - 1) // CH
RANGE = 4096
NRANGE_PER_CORE = 2
SLAB_ROWS = RANGE + 8
ZROWS = RANGE // 16
TAIL_ROWS = N - 3 * RANGE - 15 * ZROWS

_f32 = jnp.float32


def _layer_norm(x, g, b):
    m = jnp.mean(x, axis=-1, keepdims=True)
    v = jnp.mean((x - m) ** 2, axis=-1, keepdims=True)
    return (x - m) * lax.rsqrt(v + 1e-5) * g + b


def _bias3(v):
    return v.reshape(NM, 1, v.shape[-1])


def _wspec(shape):
    return pl.BlockSpec((1,) + shape, lambda m, r: (m, 0, 0))


def _enc_node(x, W1, b1, W2, b2, g, be):
    def body(x_r, W1_r, b1_r, W2_r, b2_r, g_r, be_r, o_r):
        h1 = jax.nn.silu(jnp.dot(x_r[0], W1_r[0], preferred_element_type=_f32) + b1_r[0])
        u = jnp.dot(h1, W2_r[0], preferred_element_type=_f32) + b2_r[0]
        o_r[0] = _layer_norm(u, g_r[0], be_r[0])

    return pl.pallas_call(
        body,
        grid=(NM, NRT),
        in_specs=[
            pl.BlockSpec((1, TRN, F), lambda m, r: (m, r, 0)),
            _wspec((F, D)), _wspec((1, D)), _wspec((D, D)),
            _wspec((1, D)), _wspec((1, D)), _wspec((1, D)),
        ],
        out_specs=pl.BlockSpec((1, TRN, D), lambda m, r: (m, r, 0)),
        out_shape=jax.ShapeDtypeStruct((NM, N, D), _f32),
    )(x, W1, _bias3(b1), W2, _bias3(b2), _bias3(g), _bias3(be))


def _enc_edge(ea, W1, b1, W2, b2, g, be):
    def body(x_r, W1_r, b1_r, W2_r, b2_r, g_r, be_r, o_r):
        h1 = jax.nn.silu(jnp.dot(x_r[...], W1_r[0], preferred_element_type=_f32) + b1_r[0])
        u = jnp.dot(h1, W2_r[0], preferred_element_type=_f32) + b2_r[0]
        o_r[0] = _layer_norm(u, g_r[0], be_r[0])

    return pl.pallas_call(
        body,
        grid=(NM, ERT),
        in_specs=[
            pl.BlockSpec((TRE, 4), lambda m, r: (r, 0)),
            _wspec((4, D)), _wspec((1, D)), _wspec((D, D)),
            _wspec((1, D)), _wspec((1, D)), _wspec((1, D)),
        ],
        out_specs=pl.BlockSpec((1, TRE, D), lambda m, r: (m, r, 0)),
        out_shape=jax.ShapeDtypeStruct((NM, E, D), _f32),
    )(ea, W1, _bias3(b1), W2, _bias3(b2), _bias3(g), _bias3(be))


def _proj(h, Ws, Wd):
    def body(h_r, Ws_r, Wd_r, p_r, q_r):
        hb = h_r[0]
        p_r[0] = jnp.dot(hb, Ws_r[0], preferred_element_type=_f32)
        q_r[0] = jnp.dot(hb, Wd_r[0], preferred_element_type=_f32)

    spec = pl.BlockSpec((1, TRN, D), lambda m, r: (m, r, 0))
    return pl.pallas_call(
        body,
        grid=(NM, NRT),
        in_specs=[spec, _wspec((D, D)), _wspec((D, D))],
        out_specs=[spec, spec],
        out_shape=[jax.ShapeDtypeStruct((NM, N, D), _f32)] * 2,
    )(h, Ws, Wd)


def _edge_update(gs, gd, e, W1e, b1, W2, b2, g, be):
    def body(gs_r, gd_r, e_r, W1e_r, b1_r, W2_r, b2_r, g_r, be_r, o_r):
        eb = e_r[0]
        t = jax.nn.silu(gs_r[0] + gd_r[0]
                        + jnp.dot(eb, W1e_r[0], preferred_element_type=_f32) + b1_r[0])
        u = jnp.dot(t, W2_r[0], preferred_element_type=_f32) + b2_r[0]
        o_r[0] = eb + _layer_norm(u, g_r[0], be_r[0])

    spec = pl.BlockSpec((1, TRE, D), lambda m, r: (m, r, 0))
    return pl.pallas_call(
        body,
        grid=(NM, ERT),
        in_specs=[spec, spec, spec, _wspec((D, D)), _wspec((1, D)),
                  _wspec((D, D)), _wspec((1, D)), _wspec((1, D)), _wspec((1, D))],
        out_specs=spec,
        out_shape=jax.ShapeDtypeStruct((NM, E, D), _f32),
    )(gs, gd, e, W1e, _bias3(b1), W2, _bias3(b2), _bias3(g), _bias3(be))


def _node_update(h, agg, W1h, W1a, b1, W2, b2, g, be):
    def body(h_r, a_r, W1h_r, W1a_r, b1_r, W2_r, b2_r, g_r, be_r, o_r):
        hb = h_r[0]
        t = jax.nn.silu(jnp.dot(hb, W1h_r[0], preferred_element_type=_f32)
                        + jnp.dot(a_r[0], W1a_r[0], preferred_element_type=_f32)
                        + b1_r[0])
        u = jnp.dot(t, W2_r[0], preferred_element_type=_f32) + b2_r[0]
        o_r[0] = hb + _layer_norm(u, g_r[0], be_r[0])

    spec = pl.BlockSpec((1, TRN, D), lambda m, r: (m, r, 0))
    return pl.pallas_call(
        body,
        grid=(NM, NRT),
        in_specs=[spec, spec, _wspec((D, D)), _wspec((D, D)), _wspec((1, D)),
                  _wspec((D, D)), _wspec((1, D)), _wspec((1, D)), _wspec((1, D))],
        out_specs=spec,
        out_shape=jax.ShapeDtypeStruct((NM, N, D), _f32),
    )(h, agg, W1h, W1a, _bias3(b1), W2, _bias3(b2), _bias3(g), _bias3(be))


def _decoder(h, W1, b1, W2, b2):
    def body(h_r, W1_r, b1_r, W2_r, b2_r, o_r):
        m = pl.program_id(1)
        t = jax.nn.silu(jnp.dot(h_r[0], W1_r[0], preferred_element_type=_f32) + b1_r[0])
        z = jnp.dot(t, W2_r[0], preferred_element_type=_f32) + b2_r[0]

        @pl.when(m == 0)
        def _():
            o_r[...] = z

        @pl.when(m != 0)
        def _():
            o_r[...] = o_r[...] + z

    def wspec(shape):
        return pl.BlockSpec((1,) + shape, lambda r, m: (m, 0, 0))

    return pl.pallas_call(
        body,
        grid=(NRT, NM),
        in_specs=[
            pl.BlockSpec((1, TRN, D), lambda r, m: (m, r, 0)),
            wspec((D, DEC)), wspec((1, DEC)), wspec((DEC, F)), wspec((1, F)),
        ],
        out_specs=pl.BlockSpec((TRN, F), lambda r, m: (r, 0)),
        out_shape=jax.ShapeDtypeStruct((N, F), _f32),
    )(h, W1, _bias3(b1), W2, _bias3(b2))


def _sc_mesh():
    return plsc.VectorSubcoreMesh(core_axis_name="c", subcore_axis_name="s")


def _sc_gather(Pf, Qf, src, dst):
    @functools.partial(
        pl.kernel, mesh=_sc_mesh(),
        out_type=[jax.ShapeDtypeStruct((NM, E, D), _f32),
                  jax.ShapeDtypeStruct((NM, E, D), _f32)],
        scratch_types=[
            pltpu.VMEM((CH,), jnp.int32), pltpu.VMEM((CH, D), _f32),
            pltpu.VMEM((CH,), jnp.int32), pltpu.VMEM((CH, D), _f32),
            pltpu.SemaphoreType.DMA, pltpu.SemaphoreType.DMA,
        ],
    )
    def k(P_h, Q_h, src_h, dst_h, gs_h, gd_h, idx1, rows1, idx2, rows2, sem1, sem2):
        cid = lax.axis_index("c")
        sid = lax.axis_index("s")
        wid = sid * 2 + cid
        for m in range(NM):
            def body(kk, carry, m=m):
                g = wid + kk * 32

                @pl.when(g < NCHUNK)
                def _():
                    eff = jnp.minimum(g * CH, E - CH)
                    pltpu.sync_copy(src_h.at[pl.ds(eff, CH)], idx1)
                    pltpu.sync_copy(dst_h.at[pl.ds(eff, CH)], idx2)
                    if m > 0:
                        for j in range(CH // 16):
                            sl = pl.ds(j * 16, 16)
                            idx1[sl] = idx1[sl] + (m * N)
                            idx2[sl] = idx2[sl] + (m * N)
                    cp1 = pltpu.async_copy(P_h.at[idx1], rows1, sem1)
                    cp2 = pltpu.async_copy(Q_h.at[idx2], rows2, sem2)
                    cp1.wait()
                    cp2.wait()
                    pltpu.sync_copy(rows1, gs_h.at[m, pl.ds(eff, CH)])
                    pltpu.sync_copy(rows2, gd_h.at[m, pl.ds(eff, CH)])

                return carry

            lax.fori_loop(0, 32, body, 0)

    return k(Pf, Qf, src, dst)


def _sc_segsum(e, dst, zeros):
    @functools.partial(
        pl.kernel, mesh=_sc_mesh(),
        out_type=jax.ShapeDtypeStruct((NM, N, D), _f32),
        scratch_types=[
            pltpu.VMEM((CH,), jnp.int32),
            pltpu.VMEM((CH, D), _f32),
            pltpu.VMEM((ZROWS, D), _f32),
            pltpu.VMEM_SHARED((SLAB_ROWS, D), _f32),
            pltpu.SemaphoreType.DMA,
        ],
    )
    def k(e_h, dst_h, z_h, agg_h, idxv, rows, zbuf, slab, sem):
        cid = lax.axis_index("c")
        sid = lax.axis_index("s")
        pltpu.sync_copy(z_h, zbuf)
        for m in range(NM):
            for rr in range(NRANGE_PER_CORE):
                rng = cid + 2 * rr
                base_node = rng * RANGE
                nrows = (RANGE if rr == 0
                         else jnp.where(cid == 0, RANGE, N - 3 * RANGE))
                pltpu.sync_copy(zbuf, slab.at[pl.ds(sid * ZROWS, ZROWS)])
                plsc.subcore_barrier()

                def body(kk, carry, m=m):
                    g = sid + kk * 16

                    @pl.when(g < NCHUNK)
                    def _():
                        raw = g * CH
                        eff = jnp.minimum(raw, E - CH)
                        vfrom = raw - eff
                        pltpu.sync_copy(dst_h.at[pl.ds(eff, CH)], idxv)
                        for j in range(CH // 16):
                            sl = pl.ds(j * 16, 16)
                            li = idxv[sl] - base_node
                            pos = lax.iota(jnp.int32, 16) + (j * 16)
                            ok = (li >= 0) & (li < nrows) & (pos >= vfrom)
                            idxv[sl] = jnp.where(ok, li, RANGE)
                        pltpu.sync_copy(e_h.at[m, pl.ds(eff, CH)], rows)
                        pltpu.sync_copy(rows, slab.at[idxv], add=True)

                    return carry

                lax.fori_loop(0, 64, body, 0)
                plsc.subcore_barrier()
                out_base = base_node + sid * ZROWS
                if rr == 0:
                    pltpu.sync_copy(slab.at[pl.ds(sid * ZROWS, ZROWS)],
                                    agg_h.at[m, pl.ds(out_base, ZROWS)])
                else:
                    @pl.when((cid == 0) | (sid < 15))
                    def _(m=m):
                        pltpu.sync_copy(slab.at[pl.ds(sid * ZROWS, ZROWS)],
                                        agg_h.at[m, pl.ds(out_base, ZROWS)])

                    @pl.when((cid == 1) & (sid == 15))
                    def _(m=m):
                        pltpu.sync_copy(slab.at[pl.ds(sid * ZROWS, TAIL_ROWS)],
                                        agg_h.at[m, pl.ds(out_base, TAIL_ROWS)])

                plsc.subcore_barrier()

    return k(e, dst, zeros)


def kernel(features, edge_index, edge_attr, params):
    p = params
    src = edge_index[0]
    dst = edge_index[1]
    x = features[0]

    beW1 = p["be_W1"]
    W1s, W1d, W1e = beW1[:, :, :D], beW1[:, :, D:2 * D], beW1[:, :, 2 * D:]
    bnW1 = p["bn_W1"]
    W1h, W1a = bnW1[:, :, :D], bnW1[:, :, D:]

    zeros = jnp.zeros((ZROWS, D), _f32)

    h = _enc_node(x, p["en_W1"], p["en_b1"], p["en_W2"], p["en_b2"], p["en_g"], p["en_be"])
    e = _enc_edge(edge_attr, p["ee_W1"], p["ee_b1"], p["ee_W2"], p["ee_b2"], p["ee_g"], p["ee_be"])

    for b in range(NB):
        P, Q = _proj(h, W1s[:, b], W1d[:, b])
        gs, gd = _sc_gather(P.reshape(NM * N, D), Q.reshape(NM * N, D), src, dst)
        e = _edge_update(gs, gd, e, W1e[:, b], p["be_b1"][:, b], p["be_W2"][:, b],
                         p["be_b2"][:, b], p["be_g"][:, b], p["be_be"][:, b])
        agg = _sc_segsum(e, dst, zeros)
        h = _node_update(h, agg, W1h[:, b], W1a[:, b], p["bn_b1"][:, b], p["bn_W2"][:, b],
                         p["bn_b2"][:, b], p["bn_g"][:, b], p["bn_be"][:, b])

    sw = p["step_w"]
    return _decoder(h, p["de_W1"], p["de_b1"],
                    p["de_W2"] * sw[:, None, None], p["de_b2"] * sw[:, None])


# speedup vs baseline: 1.5532x; 1.1494x over previous
import functools

import jax
import jax.numpy as jnp
from jax import lax
from jax.experimental import pallas as pl
from jax.experimental.pallas import tpu as pltpu
from jax.experimental.pallas import tpu_sc as plsc

N = 16200
E = N * 8
F = 42
D = 128
HD = D // 2
DEC = 64
NB = 3
NM = 3

TRN = 1800
NRT = N // TRN
TRE = 1600
ERT = E // TRE

CH = 128
NCHUNK = (E + CH - 1) // CH
HALF = 8192
SLAB_ROWS = HALF + 8
ZROWS = 512
TAIL_ROWS = N - HALF - 15 * ZROWS
KMAX = (NCHUNK + 15) // 16

_f32 = jnp.float32


def _layer_norm(x, g, b):
    m = jnp.mean(x, axis=-1, keepdims=True)
    v = jnp.mean((x - m) ** 2, axis=-1, keepdims=True)
    return (x - m) * lax.rsqrt(v + 1e-5) * g + b


def _bias3(v):
    return v.reshape(NM, 1, v.shape[-1])


def _wspec(shape):
    return pl.BlockSpec((1,) + shape, lambda m, r: (m, 0, 0))


def _espec_halves():
    sp = pl.BlockSpec((1, TRE, HD), lambda m, r: (m, r, 0))
    sh = [jax.ShapeDtypeStruct((NM, E, HD), _f32)] * 2
    return [sp, sp], sh


def _enc_node(x, W1, b1, W2, b2, g, be):
    def body(x_r, W1_r, b1_r, W2_r, b2_r, g_r, be_r, o_r):
        h1 = jax.nn.silu(jnp.dot(x_r[0], W1_r[0], preferred_element_type=_f32) + b1_r[0])
        u = jnp.dot(h1, W2_r[0], preferred_element_type=_f32) + b2_r[0]
        o_r[0] = _layer_norm(u, g_r[0], be_r[0])

    return pl.pallas_call(
        body,
        grid=(NM, NRT),
        in_specs=[
            pl.BlockSpec((1, TRN, F), lambda m, r: (m, r, 0)),
            _wspec((F, D)), _wspec((1, D)), _wspec((D, D)),
            _wspec((1, D)), _wspec((1, D)), _wspec((1, D)),
        ],
        out_specs=pl.BlockSpec((1, TRN, D), lambda m, r: (m, r, 0)),
        out_shape=jax.ShapeDtypeStruct((NM, N, D), _f32),
    )(x, W1, _bias3(b1), W2, _bias3(b2), _bias3(g), _bias3(be))


def _enc_edge(ea, W1, b1, W2, b2, g, be):
    def body(x_r, W1_r, b1_r, W2_r, b2_r, g_r, be_r, o0_r, o1_r):
        h1 = jax.nn.silu(jnp.dot(x_r[...], W1_r[0], preferred_element_type=_f32) + b1_r[0])
        u = jnp.dot(h1, W2_r[0], preferred_element_type=_f32) + b2_r[0]
        res = _layer_norm(u, g_r[0], be_r[0])
        o0_r[0] = res[:, :HD]
        o1_r[0] = res[:, HD:]

    out_specs, out_shape = _espec_halves()
    return pl.pallas_call(
        body,
        grid=(NM, ERT),
        in_specs=[
            pl.BlockSpec((TRE, 4), lambda m, r: (r, 0)),
            _wspec((4, D)), _wspec((1, D)), _wspec((D, D)),
            _wspec((1, D)), _wspec((1, D)), _wspec((1, D)),
        ],
        out_specs=out_specs,
        out_shape=out_shape,
    )(ea, W1, _bias3(b1), W2, _bias3(b2), _bias3(g), _bias3(be))


def _proj(h, Ws, Wd):
    def body(h_r, Ws_r, Wd_r, p_r, q_r):
        hb = h_r[0]
        p_r[0] = jnp.dot(hb, Ws_r[0], preferred_element_type=_f32)
        q_r[0] = jnp.dot(hb, Wd_r[0], preferred_element_type=_f32)

    spec = pl.BlockSpec((1, TRN, D), lambda m, r: (m, r, 0))
    return pl.pallas_call(
        body,
        grid=(NM, NRT),
        in_specs=[spec, _wspec((D, D)), _wspec((D, D))],
        out_specs=[spec, spec],
        out_shape=[jax.ShapeDtypeStruct((NM, N, D), _f32)] * 2,
    )(h, Ws, Wd)


def _edge_update(gs, gd, e0, e1, W1e, b1, W2, b2, g, be):
    def body(gs_r, gd_r, e0_r, e1_r, W1e_r, b1_r, W2_r, b2_r, g_r, be_r, o0_r, o1_r):
        eb = jnp.concatenate([e0_r[0], e1_r[0]], axis=-1)
        t = jax.nn.silu(gs_r[0] + gd_r[0]
                        + jnp.dot(eb, W1e_r[0], preferred_element_type=_f32) + b1_r[0])
        u = jnp.dot(t, W2_r[0], preferred_element_type=_f32) + b2_r[0]
        res = eb + _layer_norm(u, g_r[0], be_r[0])
        o0_r[0] = res[:, :HD]
        o1_r[0] = res[:, HD:]

    spec = pl.BlockSpec((1, TRE, D), lambda m, r: (m, r, 0))
    hspec = pl.BlockSpec((1, TRE, HD), lambda m, r: (m, r, 0))
    out_specs, out_shape = _espec_halves()
    return pl.pallas_call(
        body,
        grid=(NM, ERT),
        in_specs=[spec, spec, hspec, hspec, _wspec((D, D)), _wspec((1, D)),
                  _wspec((D, D)), _wspec((1, D)), _wspec((1, D)), _wspec((1, D))],
        out_specs=out_specs,
        out_shape=out_shape,
    )(gs, gd, e0, e1, W1e, _bias3(b1), W2, _bias3(b2), _bias3(g), _bias3(be))


def _node_update(h, agg0, agg1, W1h, W1a0, W1a1, b1, W2, b2, g, be):
    def body(h_r, a0_r, a1_r, W1h_r, Wa0_r, Wa1_r, b1_r, W2_r, b2_r, g_r, be_r, o_r):
        hb = h_r[0]
        t = jax.nn.silu(jnp.dot(hb, W1h_r[0], preferred_element_type=_f32)
                        + jnp.dot(a0_r[0], Wa0_r[0], preferred_element_type=_f32)
                        + jnp.dot(a1_r[0], Wa1_r[0], preferred_element_type=_f32)
                        + b1_r[0])
        u = jnp.dot(t, W2_r[0], preferred_element_type=_f32) + b2_r[0]
        o_r[0] = hb + _layer_norm(u, g_r[0], be_r[0])

    spec = pl.BlockSpec((1, TRN, D), lambda m, r: (m, r, 0))
    aspec = pl.BlockSpec((1, TRN, HD), lambda m, r: (m, r, 0))
    return pl.pallas_call(
        body,
        grid=(NM, NRT),
        in_specs=[spec, aspec, aspec, _wspec((D, D)), _wspec((HD, D)), _wspec((HD, D)),
                  _wspec((1, D)), _wspec((D, D)), _wspec((1, D)), _wspec((1, D)), _wspec((1, D))],
        out_specs=spec,
        out_shape=jax.ShapeDtypeStruct((NM, N, D), _f32),
    )(h, agg0, agg1, W1h, W1a0, W1a1, _bias3(b1), W2, _bias3(b2), _bias3(g), _bias3(be))


def _decoder(h, W1, b1, W2, b2):
    def body(h_r, W1_r, b1_r, W2_r, b2_r, o_r):
        m = pl.program_id(1)
        t = jax.nn.silu(jnp.dot(h_r[0], W1_r[0], preferred_element_type=_f32) + b1_r[0])
        z = jnp.dot(t, W2_r[0], preferred_element_type=_f32) + b2_r[0]

        @pl.when(m == 0)
        def _():
            o_r[...] = z

        @pl.when(m != 0)
        def _():
            o_r[...] = o_r[...] + z

    def wspec(shape):
        return pl.BlockSpec((1,) + shape, lambda r, m: (m, 0, 0))

    return pl.pallas_call(
        body,
        grid=(NRT, NM),
        in_specs=[
            pl.BlockSpec((1, TRN, D), lambda r, m: (m, r, 0)),
            wspec((D, DEC)), wspec((1, DEC)), wspec((DEC, F)), wspec((1, F)),
        ],
        out_specs=pl.BlockSpec((TRN, F), lambda r, m: (r, 0)),
        out_shape=jax.ShapeDtypeStruct((N, F), _f32),
    )(h, W1, _bias3(b1), W2, _bias3(b2))


def _sc_mesh():
    return plsc.VectorSubcoreMesh(core_axis_name="c", subcore_axis_name="s")


def _sc_gather(Pf, Qf, src, dst):
    @functools.partial(
        pl.kernel, mesh=_sc_mesh(),
        out_type=[jax.ShapeDtypeStruct((NM, E, D), _f32),
                  jax.ShapeDtypeStruct((NM, E, D), _f32)],
        scratch_types=[
            pltpu.VMEM((CH,), jnp.int32), pltpu.VMEM((CH, D), _f32),
            pltpu.VMEM((CH,), jnp.int32), pltpu.VMEM((CH, D), _f32),
            pltpu.SemaphoreType.DMA, pltpu.SemaphoreType.DMA,
        ],
    )
    def k(P_h, Q_h, src_h, dst_h, gs_h, gd_h, idx1, rows1, idx2, rows2, sem1, sem2):
        cid = lax.axis_index("c")
        sid = lax.axis_index("s")
        wid = sid * 2 + cid
        for m in range(NM):
            def body(kk, carry, m=m):
                g = wid + kk * 32

                @pl.when(g < NCHUNK)
                def _():
                    eff = jnp.minimum(g * CH, E - CH)
                    pltpu.sync_copy(src_h.at[pl.ds(eff, CH)], idx1)
                    pltpu.sync_copy(dst_h.at[pl.ds(eff, CH)], idx2)
                    if m > 0:
                        for j in range(CH // 16):
                            sl = pl.ds(j * 16, 16)
                            idx1[sl] = idx1[sl] + (m * N)
                            idx2[sl] = idx2[sl] + (m * N)
                    cp1 = pltpu.async_copy(P_h.at[idx1], rows1, sem1)
                    cp2 = pltpu.async_copy(Q_h.at[idx2], rows2, sem2)
                    cp1.wait()
                    cp2.wait()
                    pltpu.sync_copy(rows1, gs_h.at[m, pl.ds(eff, CH)])
                    pltpu.sync_copy(rows2, gd_h.at[m, pl.ds(eff, CH)])

                return carry

            lax.fori_loop(0, 32, body, 0)

    return k(Pf, Qf, src, dst)


def _sc_segsum(e0, e1, dst, zeros):
    @functools.partial(
        pl.kernel, mesh=_sc_mesh(),
        out_type=[jax.ShapeDtypeStruct((NM, N, HD), _f32),
                  jax.ShapeDtypeStruct((NM, N, HD), _f32)],
        scratch_types=[
            pltpu.VMEM((CH,), jnp.int32),
            pltpu.VMEM((CH,), jnp.int32),
            pltpu.VMEM((CH, HD), _f32),
            pltpu.VMEM((CH, HD), _f32),
            pltpu.VMEM((ZROWS // 8, HD), _f32),
            pltpu.VMEM_SHARED((SLAB_ROWS, HD), _f32),
            pltpu.SemaphoreType.DMA,
            pltpu.SemaphoreType.DMA,
        ],
    )
    def k(e0_h, e1_h, dst_h, z_h, a0_h, a1_h,
          idxA, idxB, rows0, rows1, zbuf, slab, semA, semB):
        cid = lax.axis_index("c")
        sid = lax.axis_index("s")
        base_node = cid * HALF
        nrows = jnp.where(cid == 0, HALF, N - HALF)
        pltpu.sync_copy(z_h, zbuf)

        def eff_of(g):
            return jnp.minimum(g * CH, E - CH)

        def load_remap(g, idxv):
            raw = g * CH
            eff = eff_of(g)
            vfrom = raw - eff
            pltpu.sync_copy(dst_h.at[pl.ds(eff, CH)], idxv)
            for j in range(CH // 16):
                sl = pl.ds(j * 16, 16)
                li = idxv[sl] - base_node
                pos = lax.iota(jnp.int32, 16) + (j * 16)
                ok = (li >= 0) & (li < nrows) & (pos >= vfrom)
                idxv[sl] = jnp.where(ok, li, HALF)

        for m in range(NM):
            for e_h, agg_h in ((e0_h, a0_h), (e1_h, a1_h)):
                for z in range(8):
                    pltpu.sync_copy(zbuf, slab.at[pl.ds(sid * ZROWS + z * (ZROWS // 8),
                                                        ZROWS // 8)])
                plsc.subcore_barrier()

                pltpu.async_copy(e_h.at[m, pl.ds(eff_of(sid), CH)], rows0, semA)

                def pair(q, carry, m=m, e_h=e_h):
                    gA = sid + 32 * q
                    gB = gA + 16
                    gA2 = gA + 32

                    @pl.when(gB < NCHUNK)
                    def _():
                        pltpu.async_copy(e_h.at[m, pl.ds(eff_of(gB), CH)], rows1, semB)

                    load_remap(gA, idxA)
                    pltpu.make_async_copy(e_h.at[m, pl.ds(eff_of(gA), CH)], rows0, semA).wait()
                    pltpu.sync_copy(rows0, slab.at[idxA], add=True)

                    @pl.when(gA2 < NCHUNK)
                    def _():
                        pltpu.async_copy(e_h.at[m, pl.ds(eff_of(gA2), CH)], rows0, semA)

                    @pl.when(gB < NCHUNK)
                    def _():
                        load_remap(gB, idxB)
                        pltpu.make_async_copy(e_h.at[m, pl.ds(eff_of(gB), CH)], rows1, semB).wait()
                        pltpu.sync_copy(rows1, slab.at[idxB], add=True)

                    return carry

                lax.fori_loop(0, KMAX // 2, pair, 0)
                plsc.subcore_barrier()
                out_base = base_node + sid * ZROWS

                @pl.when((cid == 0) | (sid < 15))
                def _(m=m, agg_h=agg_h):
                    pltpu.sync_copy(slab.at[pl.ds(sid * ZROWS, ZROWS)],
                                    agg_h.at[m, pl.ds(out_base, ZROWS)])

                @pl.when((cid == 1) & (sid == 15))
                def _(m=m, agg_h=agg_h):
                    pltpu.sync_copy(slab.at[pl.ds(sid * ZROWS, TAIL_ROWS)],
                                    agg_h.at[m, pl.ds(out_base, TAIL_ROWS)])

                plsc.subcore_barrier()

    return k(e0, e1, dst, zeros)


def kernel(features, edge_index, edge_attr, params):
    p = params
    src = edge_index[0]
    dst = edge_index[1]
    x = features[0]

    beW1 = p["be_W1"]
    W1s, W1d, W1e = beW1[:, :, :D], beW1[:, :, D:2 * D], beW1[:, :, 2 * D:]
    bnW1 = p["bn_W1"]
    W1h = bnW1[:, :, :D]
    W1a0 = bnW1[:, :, D:D + HD]
    W1a1 = bnW1[:, :, D + HD:]

    zeros = jnp.zeros((ZROWS // 8, HD), _f32)

    h = _enc_node(x, p["en_W1"], p["en_b1"], p["en_W2"], p["en_b2"], p["en_g"], p["en_be"])
    e0, e1 = _enc_edge(edge_attr, p["ee_W1"], p["ee_b1"], p["ee_W2"], p["ee_b2"],
                       p["ee_g"], p["ee_be"])

    for b in range(NB):
        P, Q = _proj(h, W1s[:, b], W1d[:, b])
        gs, gd = _sc_gather(P.reshape(NM * N, D), Q.reshape(NM * N, D), src, dst)
        e0, e1 = _edge_update(gs, gd, e0, e1, W1e[:, b], p["be_b1"][:, b], p["be_W2"][:, b],
                              p["be_b2"][:, b], p["be_g"][:, b], p["be_be"][:, b])
        agg0, agg1 = _sc_segsum(e0, e1, dst, zeros)
        h = _node_update(h, agg0, agg1, W1h[:, b], W1a0[:, b], W1a1[:, b], p["bn_b1"][:, b],
                         p["bn_W2"][:, b], p["bn_b2"][:, b], p["bn_g"][:, b], p["bn_be"][:, b])

    sw = p["step_w"]
    return _decoder(h, p["de_W1"], p["de_b1"],
                    p["de_W2"] * sw[:, None, None], p["de_b2"] * sw[:, None])


# speedup vs baseline: 1.5771x; 1.0154x over previous
import functools

import jax
import jax.numpy as jnp
from jax import lax
from jax.experimental import pallas as pl
from jax.experimental.pallas import tpu as pltpu
from jax.experimental.pallas import tpu_sc as plsc

N = 16200
E = N * 8
F = 42
D = 128
HD = D // 2
DEC = 64
NB = 3
NM = 3

TRN = 1800
NRT = N // TRN
TRE = 1600
ERT = E // TRE

CH = 128
NCHUNK = (E + CH - 1) // CH
BIG = 1024
SUBE = 8192
NBIG = SUBE // BIG
HALF = 8192
SLAB_ROWS = HALF + 8
ZROWS = 512
TAIL_ROWS = N - HALF - 15 * ZROWS
KMAX = (NCHUNK + 15) // 16

_f32 = jnp.float32


def _layer_norm(x, g, b):
    m = jnp.mean(x, axis=-1, keepdims=True)
    v = jnp.mean((x - m) ** 2, axis=-1, keepdims=True)
    return (x - m) * lax.rsqrt(v + 1e-5) * g + b


def _bias3(v):
    return v.reshape(NM, 1, v.shape[-1])


def _wspec(shape):
    return pl.BlockSpec((1,) + shape, lambda m, r: (m, 0, 0))


def _espec_halves():
    sp = pl.BlockSpec((1, TRE, HD), lambda m, r: (m, r, 0))
    sh = [jax.ShapeDtypeStruct((NM, E, HD), _f32)] * 2
    return [sp, sp], sh


def _enc_node(x, W1, b1, W2, b2, g, be, Ws, Wd):
    def body(x_r, W1_r, b1_r, W2_r, b2_r, g_r, be_r, Ws_r, Wd_r, o_r, p_r, q_r):
        h1 = jax.nn.silu(jnp.dot(x_r[0], W1_r[0], preferred_element_type=_f32) + b1_r[0])
        u = jnp.dot(h1, W2_r[0], preferred_element_type=_f32) + b2_r[0]
        hb = _layer_norm(u, g_r[0], be_r[0])
        o_r[0] = hb
        p_r[0] = jnp.dot(hb, Ws_r[0], preferred_element_type=_f32)
        q_r[0] = jnp.dot(hb, Wd_r[0], preferred_element_type=_f32)

    spec = pl.BlockSpec((1, TRN, D), lambda m, r: (m, r, 0))
    return pl.pallas_call(
        body,
        grid=(NM, NRT),
        in_specs=[
            pl.BlockSpec((1, TRN, F), lambda m, r: (m, r, 0)),
            _wspec((F, D)), _wspec((1, D)), _wspec((D, D)),
            _wspec((1, D)), _wspec((1, D)), _wspec((1, D)),
            _wspec((D, D)), _wspec((D, D)),
        ],
        out_specs=[spec, spec, spec],
        out_shape=[jax.ShapeDtypeStruct((NM, N, D), _f32)] * 3,
    )(x, W1, _bias3(b1), W2, _bias3(b2), _bias3(g), _bias3(be), Ws, Wd)


def _enc_edge(ea, W1, b1, W2, b2, g, be):
    def body(x_r, W1_r, b1_r, W2_r, b2_r, g_r, be_r, o0_r, o1_r):
        h1 = jax.nn.silu(jnp.dot(x_r[...], W1_r[0], preferred_element_type=_f32) + b1_r[0])
        u = jnp.dot(h1, W2_r[0], preferred_element_type=_f32) + b2_r[0]
        res = _layer_norm(u, g_r[0], be_r[0])
        o0_r[0] = res[:, :HD]
        o1_r[0] = res[:, HD:]

    out_specs, out_shape = _espec_halves()
    return pl.pallas_call(
        body,
        grid=(NM, ERT),
        in_specs=[
            pl.BlockSpec((TRE, 4), lambda m, r: (r, 0)),
            _wspec((4, D)), _wspec((1, D)), _wspec((D, D)),
            _wspec((1, D)), _wspec((1, D)), _wspec((1, D)),
        ],
        out_specs=out_specs,
        out_shape=out_shape,
    )(ea, W1, _bias3(b1), W2, _bias3(b2), _bias3(g), _bias3(be))


def _node_body_core(h_r, a0_r, a1_r, W1h_r, Wa0_r, Wa1_r, b1_r, W2_r, b2_r, g_r, be_r):
    hb = h_r[0]
    t = jax.nn.silu(jnp.dot(hb, W1h_r[0], preferred_element_type=_f32)
                    + jnp.dot(a0_r[0], Wa0_r[0], preferred_element_type=_f32)
                    + jnp.dot(a1_r[0], Wa1_r[0], preferred_element_type=_f32)
                    + b1_r[0])
    u = jnp.dot(t, W2_r[0], preferred_element_type=_f32) + b2_r[0]
    return hb + _layer_norm(u, g_r[0], be_r[0])


def _node_update_proj(h, agg0, agg1, W1h, W1a0, W1a1, b1, W2, b2, g, be, Ws, Wd):
    def body(h_r, a0_r, a1_r, W1h_r, Wa0_r, Wa1_r, b1_r, W2_r, b2_r, g_r, be_r,
             Ws_r, Wd_r, o_r, p_r, q_r):
        hn = _node_body_core(h_r, a0_r, a1_r, W1h_r, Wa0_r, Wa1_r, b1_r, W2_r,
                             b2_r, g_r, be_r)
        o_r[0] = hn
        p_r[0] = jnp.dot(hn, Ws_r[0], preferred_element_type=_f32)
        q_r[0] = jnp.dot(hn, Wd_r[0], preferred_element_type=_f32)

    spec = pl.BlockSpec((1, TRN, D), lambda m, r: (m, r, 0))
    aspec = pl.BlockSpec((1, TRN, HD), lambda m, r: (m, r, 0))
    return pl.pallas_call(
        body,
        grid=(NM, NRT),
        in_specs=[spec, aspec, aspec, _wspec((D, D)), _wspec((HD, D)), _wspec((HD, D)),
                  _wspec((1, D)), _wspec((D, D)), _wspec((1, D)), _wspec((1, D)),
                  _wspec((1, D)), _wspec((D, D)), _wspec((D, D))],
        out_specs=[spec, spec, spec],
        out_shape=[jax.ShapeDtypeStruct((NM, N, D), _f32)] * 3,
    )(h, agg0, agg1, W1h, W1a0, W1a1, _bias3(b1), W2, _bias3(b2), _bias3(g),
      _bias3(be), Ws, Wd)


def _edge_update(gs, gd, e0, e1, W1e, b1, W2, b2, g, be):
    def body(gs_r, gd_r, e0_r, e1_r, W1e_r, b1_r, W2_r, b2_r, g_r, be_r, o0_r, o1_r):
        eb = jnp.concatenate([e0_r[0], e1_r[0]], axis=-1)
        t = jax.nn.silu(gs_r[0] + gd_r[0]
                        + jnp.dot(eb, W1e_r[0], preferred_element_type=_f32) + b1_r[0])
        u = jnp.dot(t, W2_r[0], preferred_element_type=_f32) + b2_r[0]
        res = eb + _layer_norm(u, g_r[0], be_r[0])
        o0_r[0] = res[:, :HD]
        o1_r[0] = res[:, HD:]

    spec = pl.BlockSpec((1, TRE, D), lambda m, r: (m, r, 0))
    hspec = pl.BlockSpec((1, TRE, HD), lambda m, r: (m, r, 0))
    out_specs, out_shape = _espec_halves()
    return pl.pallas_call(
        body,
        grid=(NM, ERT),
        in_specs=[spec, spec, hspec, hspec, _wspec((D, D)), _wspec((1, D)),
                  _wspec((D, D)), _wspec((1, D)), _wspec((1, D)), _wspec((1, D))],
        out_specs=out_specs,
        out_shape=out_shape,
    )(gs, gd, e0, e1, W1e, _bias3(b1), W2, _bias3(b2), _bias3(g), _bias3(be))


def _node_update(h, agg0, agg1, W1h, W1a0, W1a1, b1, W2, b2, g, be):
    def body(h_r, a0_r, a1_r, W1h_r, Wa0_r, Wa1_r, b1_r, W2_r, b2_r, g_r, be_r, o_r):
        o_r[0] = _node_body_core(h_r, a0_r, a1_r, W1h_r, Wa0_r, Wa1_r, b1_r,
                                 W2_r, b2_r, g_r, be_r)

    spec = pl.BlockSpec((1, TRN, D), lambda m, r: (m, r, 0))
    aspec = pl.BlockSpec((1, TRN, HD), lambda m, r: (m, r, 0))
    return pl.pallas_call(
        body,
        grid=(NM, NRT),
        in_specs=[spec, aspec, aspec, _wspec((D, D)), _wspec((HD, D)), _wspec((HD, D)),
                  _wspec((1, D)), _wspec((D, D)), _wspec((1, D)), _wspec((1, D)), _wspec((1, D))],
        out_specs=spec,
        out_shape=jax.ShapeDtypeStruct((NM, N, D), _f32),
    )(h, agg0, agg1, W1h, W1a0, W1a1, _bias3(b1), W2, _bias3(b2), _bias3(g), _bias3(be))


def _decoder(h, W1, b1, W2, b2):
    def body(h_r, W1_r, b1_r, W2_r, b2_r, o_r):
        m = pl.program_id(1)
        t = jax.nn.silu(jnp.dot(h_r[0], W1_r[0], preferred_element_type=_f32) + b1_r[0])
        z = jnp.dot(t, W2_r[0], preferred_element_type=_f32) + b2_r[0]

        @pl.when(m == 0)
        def _():
            o_r[...] = z

        @pl.when(m != 0)
        def _():
            o_r[...] = o_r[...] + z

    def wspec(shape):
        return pl.BlockSpec((1,) + shape, lambda r, m: (m, 0, 0))

    return pl.pallas_call(
        body,
        grid=(NRT, NM),
        in_specs=[
            pl.BlockSpec((1, TRN, D), lambda r, m: (m, r, 0)),
            wspec((D, DEC)), wspec((1, DEC)), wspec((DEC, F)), wspec((1, F)),
        ],
        out_specs=pl.BlockSpec((TRN, F), lambda r, m: (r, 0)),
        out_shape=jax.ShapeDtypeStruct((N, F), _f32),
    )(h, W1, _bias3(b1), W2, _bias3(b2))


def _sc_mesh():
    return plsc.VectorSubcoreMesh(core_axis_name="c", subcore_axis_name="s")


def _sc_gather(Pf, Qf, src, dst):
    @functools.partial(
        pl.kernel, mesh=_sc_mesh(),
        out_type=[jax.ShapeDtypeStruct((NM, E, D), _f32),
                  jax.ShapeDtypeStruct((NM, E, D), _f32)],
        scratch_types=[
            pltpu.VMEM((CH,), jnp.int32), pltpu.VMEM((CH, D), _f32),
            pltpu.VMEM((CH,), jnp.int32), pltpu.VMEM((CH, D), _f32),
            pltpu.SemaphoreType.DMA, pltpu.SemaphoreType.DMA,
        ],
    )
    def k(P_h, Q_h, src_h, dst_h, gs_h, gd_h, idx1, rows1, idx2, rows2, sem1, sem2):
        cid = lax.axis_index("c")
        sid = lax.axis_index("s")
        wid = sid * 2 + cid
        for m in range(NM):
            def body(kk, carry, m=m):
                g = wid + kk * 32

                @pl.when(g < NCHUNK)
                def _():
                    eff = jnp.minimum(g * CH, E - CH)
                    pltpu.sync_copy(src_h.at[pl.ds(eff, CH)], idx1)
                    pltpu.sync_copy(dst_h.at[pl.ds(eff, CH)], idx2)
                    if m > 0:
                        for j in range(CH // 16):
                            sl = pl.ds(j * 16, 16)
                            idx1[sl] = idx1[sl] + (m * N)
                            idx2[sl] = idx2[sl] + (m * N)
                    cp1 = pltpu.async_copy(P_h.at[idx1], rows1, sem1)
                    cp2 = pltpu.async_copy(Q_h.at[idx2], rows2, sem2)
                    cp1.wait()
                    cp2.wait()
                    pltpu.sync_copy(rows1, gs_h.at[m, pl.ds(eff, CH)])
                    pltpu.sync_copy(rows2, gd_h.at[m, pl.ds(eff, CH)])

                return carry

            lax.fori_loop(0, 32, body, 0)

    return k(Pf, Qf, src, dst)


def _sc_segsum(e0, e1, dst, zeros):
    @functools.partial(
        pl.kernel, mesh=_sc_mesh(),
        out_type=[jax.ShapeDtypeStruct((NM, N, HD), _f32),
                  jax.ShapeDtypeStruct((NM, N, HD), _f32)],
        scratch_types=[
            pltpu.VMEM((BIG,), jnp.int32),
            pltpu.VMEM((CH,), jnp.int32),
            pltpu.VMEM((CH,), jnp.int32),
            pltpu.VMEM((CH, HD), _f32),
            pltpu.VMEM((CH, HD), _f32),
            pltpu.VMEM((ZROWS // 8, HD), _f32),
            pltpu.VMEM_SHARED((SLAB_ROWS, HD), _f32),
            pltpu.SemaphoreType.DMA,
            pltpu.SemaphoreType.DMA,
        ],
    )
    def k(e0_h, e1_h, dst_h, z_h, a0_h, a1_h,
          idxbig, idxA, idxB, rows0, rows1, zbuf, slab, semA, semB):
        cid = lax.axis_index("c")
        sid = lax.axis_index("s")
        base_node = cid * HALF
        nrows = jnp.where(cid == 0, HALF, N - HALF)
        pltpu.sync_copy(z_h, zbuf)

        def eff_of(g):
            return jnp.minimum(g * CH, E - CH)

        def load_remap(g, idxv):
            raw = g * CH
            eff = eff_of(g)
            vfrom = raw - eff
            pltpu.sync_copy(dst_h.at[pl.ds(eff, CH)], idxv)
            for j in range(CH // 16):
                sl = pl.ds(j * 16, 16)
                li = idxv[sl] - base_node
                pos = lax.iota(jnp.int32, 16) + (j * 16)
                ok = (li >= 0) & (li < nrows) & (pos >= vfrom)
                idxv[sl] = jnp.where(ok, li, HALF)

        for m in range(NM):
            for e_h, agg_h in ((e0_h, a0_h), (e1_h, a1_h)):
                for z in range(8):
                    pltpu.sync_copy(zbuf, slab.at[pl.ds(sid * ZROWS + z * (ZROWS // 8),
                                                        ZROWS // 8)])
                plsc.subcore_barrier()

                pltpu.async_copy(e_h.at[m, pl.ds(eff_of(sid), CH)], rows0, semA)

                def pair(q, carry, m=m, e_h=e_h):
                    gA = sid + 32 * q
                    gB = gA + 16
                    gA2 = gA + 32

                    @pl.when(gB < NCHUNK)
                    def _():
                        pltpu.async_copy(e_h.at[m, pl.ds(eff_of(gB), CH)], rows1, semB)

                    load_remap(gA, idxA)
                    pltpu.make_async_copy(e_h.at[m, pl.ds(eff_of(gA), CH)], rows0, semA).wait()
                    pltpu.sync_copy(rows0, slab.at[idxA], add=True)

                    @pl.when(gA2 < NCHUNK)
                    def _():
                        pltpu.async_copy(e_h.at[m, pl.ds(eff_of(gA2), CH)], rows0, semA)

                    @pl.when(gB < NCHUNK)
                    def _():
                        load_remap(gB, idxB)
                        pltpu.make_async_copy(e_h.at[m, pl.ds(eff_of(gB), CH)], rows1, semB).wait()
                        pltpu.sync_copy(rows1, slab.at[idxB], add=True)

                    return carry

                lax.fori_loop(0, KMAX // 2, pair, 0)
                plsc.subcore_barrier()
                out_base = base_node + sid * ZROWS

                @pl.when((cid == 0) | (sid < 15))
                def _(m=m, agg_h=agg_h):
                    pltpu.sync_copy(slab.at[pl.ds(sid * ZROWS, ZROWS)],
                                    agg_h.at[m, pl.ds(out_base, ZROWS)])

                @pl.when((cid == 1) & (sid == 15))
                def _(m=m, agg_h=agg_h):
                    pltpu.sync_copy(slab.at[pl.ds(sid * ZROWS, TAIL_ROWS)],
                                    agg_h.at[m, pl.ds(out_base, TAIL_ROWS)])

                plsc.subcore_barrier()

    return k(e0, e1, dst, zeros)


def kernel(features, edge_index, edge_attr, params):
    p = params
    src = edge_index[0]
    dst = edge_index[1]
    x = features[0]

    beW1 = p["be_W1"]
    W1s, W1d, W1e = beW1[:, :, :D], beW1[:, :, D:2 * D], beW1[:, :, 2 * D:]
    bnW1 = p["bn_W1"]
    W1h = bnW1[:, :, :D]
    W1a0 = bnW1[:, :, D:D + HD]
    W1a1 = bnW1[:, :, D + HD:]

    zeros = jnp.zeros((ZROWS // 8, HD), _f32)

    h, P, Q = _enc_node(x, p["en_W1"], p["en_b1"], p["en_W2"], p["en_b2"],
                        p["en_g"], p["en_be"], W1s[:, 0], W1d[:, 0])
    e0, e1 = _enc_edge(edge_attr, p["ee_W1"], p["ee_b1"], p["ee_W2"], p["ee_b2"],
                       p["ee_g"], p["ee_be"])

    for b in range(NB):
        gs, gd = _sc_gather(P.reshape(NM * N, D), Q.reshape(NM * N, D), src, dst)
        e0, e1 = _edge_update(gs, gd, e0, e1, W1e[:, b], p["be_b1"][:, b], p["be_W2"][:, b],
                              p["be_b2"][:, b], p["be_g"][:, b], p["be_be"][:, b])
        agg0, agg1 = _sc_segsum(e0, e1, dst, zeros)
        nargs = (h, agg0, agg1, W1h[:, b], W1a0[:, b], W1a1[:, b], p["bn_b1"][:, b],
                 p["bn_W2"][:, b], p["bn_b2"][:, b], p["bn_g"][:, b], p["bn_be"][:, b])
        if b < NB - 1:
            h, P, Q = _node_update_proj(*nargs, W1s[:, b + 1], W1d[:, b + 1])
        else:
            h = _node_update(*nargs)

    sw = p["step_w"]
    return _decoder(h, p["de_W1"], p["de_b1"],
                    p["de_W2"] * sw[:, None, None], p["de_b2"] * sw[:, None])


# speedup vs baseline: 1.6442x; 1.0426x over previous
import functools

import jax
import jax.numpy as jnp
from jax import lax
from jax.experimental import pallas as pl
from jax.experimental.pallas import tpu as pltpu
from jax.experimental.pallas import tpu_sc as plsc

N = 16200
E = N * 8
F = 42
D = 128
HD = D // 2
DEC = 64
NB = 3
NM = 3

TRN = 1800
NRT = N // TRN
TRE = 1600
ERT = E // TRE

CH = 128
NCHUNK = (E + CH - 1) // CH
BIG = 1024
SUBE = 8192
NBIG = SUBE // BIG
HALF = 8192
SLAB_ROWS = HALF + 8
ZROWS = 512
TAIL_ROWS = N - HALF - 15 * ZROWS
KMAX = (NCHUNK + 15) // 16

_f32 = jnp.float32


def _layer_norm(x, g, b):
    m = jnp.mean(x, axis=-1, keepdims=True)
    v = jnp.mean((x - m) ** 2, axis=-1, keepdims=True)
    return (x - m) * lax.rsqrt(v + 1e-5) * g + b


def _bias3(v):
    return v.reshape(NM, 1, v.shape[-1])


def _wspec(shape):
    return pl.BlockSpec((1,) + shape, lambda m, r: (m, 0, 0))


def _espec_halves():
    sp = pl.BlockSpec((1, TRE, HD), lambda m, r: (m, r, 0))
    sh = [jax.ShapeDtypeStruct((NM, E, HD), _f32)] * 2
    return [sp, sp], sh


def _enc_node(x, W1, b1, W2, b2, g, be, Ws, Wd):
    def body(x_r, W1_r, b1_r, W2_r, b2_r, g_r, be_r, Ws_r, Wd_r, o_r, p_r, q_r):
        h1 = jax.nn.silu(jnp.dot(x_r[0], W1_r[0], preferred_element_type=_f32) + b1_r[0])
        u = jnp.dot(h1, W2_r[0], preferred_element_type=_f32) + b2_r[0]
        hb = _layer_norm(u, g_r[0], be_r[0])
        o_r[0] = hb
        p_r[0] = jnp.dot(hb, Ws_r[0], preferred_element_type=_f32)
        q_r[0] = jnp.dot(hb, Wd_r[0], preferred_element_type=_f32)

    spec = pl.BlockSpec((1, TRN, D), lambda m, r: (m, r, 0))
    return pl.pallas_call(
        body,
        grid=(NM, NRT),
        in_specs=[
            pl.BlockSpec((1, TRN, F), lambda m, r: (m, r, 0)),
            _wspec((F, D)), _wspec((1, D)), _wspec((D, D)),
            _wspec((1, D)), _wspec((1, D)), _wspec((1, D)),
            _wspec((D, D)), _wspec((D, D)),
        ],
        out_specs=[spec, spec, spec],
        out_shape=[jax.ShapeDtypeStruct((NM, N, D), _f32)] * 3,
    )(x, W1, _bias3(b1), W2, _bias3(b2), _bias3(g), _bias3(be), Ws, Wd)


def _enc_edge(ea, W1, b1, W2, b2, g, be):
    def body(x_r, W1_r, b1_r, W2_r, b2_r, g_r, be_r, o0_r, o1_r):
        h1 = jax.nn.silu(jnp.dot(x_r[...], W1_r[0], preferred_element_type=_f32) + b1_r[0])
        u = jnp.dot(h1, W2_r[0], preferred_element_type=_f32) + b2_r[0]
        res = _layer_norm(u, g_r[0], be_r[0])
        o0_r[0] = res[:, :HD]
        o1_r[0] = res[:, HD:]

    out_specs, out_shape = _espec_halves()
    return pl.pallas_call(
        body,
        grid=(NM, ERT),
        in_specs=[
            pl.BlockSpec((TRE, 4), lambda m, r: (r, 0)),
            _wspec((4, D)), _wspec((1, D)), _wspec((D, D)),
            _wspec((1, D)), _wspec((1, D)), _wspec((1, D)),
        ],
        out_specs=out_specs,
        out_shape=out_shape,
    )(ea, W1, _bias3(b1), W2, _bias3(b2), _bias3(g), _bias3(be))


def _node_body_core(h_r, a0_r, a1_r, W1h_r, Wa0_r, Wa1_r, b1_r, W2_r, b2_r, g_r, be_r):
    hb = h_r[0]
    t = jax.nn.silu(jnp.dot(hb, W1h_r[0], preferred_element_type=_f32)
                    + jnp.dot(a0_r[0], Wa0_r[0], preferred_element_type=_f32)
                    + jnp.dot(a1_r[0], Wa1_r[0], preferred_element_type=_f32)
                    + b1_r[0])
    u = jnp.dot(t, W2_r[0], preferred_element_type=_f32) + b2_r[0]
    return hb + _layer_norm(u, g_r[0], be_r[0])


def _node_update_proj(h, agg0, agg1, W1h, W1a0, W1a1, b1, W2, b2, g, be, Ws, Wd):
    def body(h_r, a0_r, a1_r, W1h_r, Wa0_r, Wa1_r, b1_r, W2_r, b2_r, g_r, be_r,
             Ws_r, Wd_r, o_r, p_r, q_r):
        hn = _node_body_core(h_r, a0_r, a1_r, W1h_r, Wa0_r, Wa1_r, b1_r, W2_r,
                             b2_r, g_r, be_r)
        o_r[0] = hn
        p_r[0] = jnp.dot(hn, Ws_r[0], preferred_element_type=_f32)
        q_r[0] = jnp.dot(hn, Wd_r[0], preferred_element_type=_f32)

    spec = pl.BlockSpec((1, TRN, D), lambda m, r: (m, r, 0))
    aspec = pl.BlockSpec((1, TRN, HD), lambda m, r: (m, r, 0))
    return pl.pallas_call(
        body,
        grid=(NM, NRT),
        in_specs=[spec, aspec, aspec, _wspec((D, D)), _wspec((HD, D)), _wspec((HD, D)),
                  _wspec((1, D)), _wspec((D, D)), _wspec((1, D)), _wspec((1, D)),
                  _wspec((1, D)), _wspec((D, D)), _wspec((D, D))],
        out_specs=[spec, spec, spec],
        out_shape=[jax.ShapeDtypeStruct((NM, N, D), _f32)] * 3,
    )(h, agg0, agg1, W1h, W1a0, W1a1, _bias3(b1), W2, _bias3(b2), _bias3(g),
      _bias3(be), Ws, Wd)


def _edge_update(gs, gd, e0, e1, W1e, b1, W2, b2, g, be):
    def body(gs_r, gd_r, e0_r, e1_r, W1e_r, b1_r, W2_r, b2_r, g_r, be_r, o0_r, o1_r):
        eb = jnp.concatenate([e0_r[0], e1_r[0]], axis=-1)
        t = jax.nn.silu(gs_r[0] + gd_r[0]
                        + jnp.dot(eb, W1e_r[0], preferred_element_type=_f32) + b1_r[0])
        u = jnp.dot(t, W2_r[0], preferred_element_type=_f32) + b2_r[0]
        res = eb + _layer_norm(u, g_r[0], be_r[0])
        o0_r[0] = res[:, :HD]
        o1_r[0] = res[:, HD:]

    spec = pl.BlockSpec((1, TRE, D), lambda m, r: (m, r, 0))
    hspec = pl.BlockSpec((1, TRE, HD), lambda m, r: (m, r, 0))
    out_specs, out_shape = _espec_halves()
    return pl.pallas_call(
        body,
        grid=(NM, ERT),
        in_specs=[spec, spec, hspec, hspec, _wspec((D, D)), _wspec((1, D)),
                  _wspec((D, D)), _wspec((1, D)), _wspec((1, D)), _wspec((1, D))],
        out_specs=out_specs,
        out_shape=out_shape,
    )(gs, gd, e0, e1, W1e, _bias3(b1), W2, _bias3(b2), _bias3(g), _bias3(be))


def _node_update(h, agg0, agg1, W1h, W1a0, W1a1, b1, W2, b2, g, be):
    def body(h_r, a0_r, a1_r, W1h_r, Wa0_r, Wa1_r, b1_r, W2_r, b2_r, g_r, be_r, o_r):
        o_r[0] = _node_body_core(h_r, a0_r, a1_r, W1h_r, Wa0_r, Wa1_r, b1_r,
                                 W2_r, b2_r, g_r, be_r)

    spec = pl.BlockSpec((1, TRN, D), lambda m, r: (m, r, 0))
    aspec = pl.BlockSpec((1, TRN, HD), lambda m, r: (m, r, 0))
    return pl.pallas_call(
        body,
        grid=(NM, NRT),
        in_specs=[spec, aspec, aspec, _wspec((D, D)), _wspec((HD, D)), _wspec((HD, D)),
                  _wspec((1, D)), _wspec((D, D)), _wspec((1, D)), _wspec((1, D)), _wspec((1, D))],
        out_specs=spec,
        out_shape=jax.ShapeDtypeStruct((NM, N, D), _f32),
    )(h, agg0, agg1, W1h, W1a0, W1a1, _bias3(b1), W2, _bias3(b2), _bias3(g), _bias3(be))


def _decoder(h, W1, b1, W2, b2):
    def body(h_r, W1_r, b1_r, W2_r, b2_r, o_r):
        m = pl.program_id(1)
        t = jax.nn.silu(jnp.dot(h_r[0], W1_r[0], preferred_element_type=_f32) + b1_r[0])
        z = jnp.dot(t, W2_r[0], preferred_element_type=_f32) + b2_r[0]

        @pl.when(m == 0)
        def _():
            o_r[...] = z

        @pl.when(m != 0)
        def _():
            o_r[...] = o_r[...] + z

    def wspec(shape):
        return pl.BlockSpec((1,) + shape, lambda r, m: (m, 0, 0))

    return pl.pallas_call(
        body,
        grid=(NRT, NM),
        in_specs=[
            pl.BlockSpec((1, TRN, D), lambda r, m: (m, r, 0)),
            wspec((D, DEC)), wspec((1, DEC)), wspec((DEC, F)), wspec((1, F)),
        ],
        out_specs=pl.BlockSpec((TRN, F), lambda r, m: (r, 0)),
        out_shape=jax.ShapeDtypeStruct((N, F), _f32),
    )(h, W1, _bias3(b1), W2, _bias3(b2))


def _sc_mesh():
    return plsc.VectorSubcoreMesh(core_axis_name="c", subcore_axis_name="s")


def _sc_gather(Pf, Qf, src, dst):
    @functools.partial(
        pl.kernel, mesh=_sc_mesh(),
        out_type=[jax.ShapeDtypeStruct((NM, E, D), _f32),
                  jax.ShapeDtypeStruct((NM, E, D), _f32)],
        scratch_types=[
            pltpu.VMEM((CH,), jnp.int32), pltpu.VMEM((CH,), jnp.int32),
            pltpu.VMEM((CH,), jnp.int32), pltpu.VMEM((CH,), jnp.int32),
            pltpu.VMEM((CH, D), _f32), pltpu.VMEM((CH, D), _f32),
            pltpu.VMEM((CH, D), _f32), pltpu.VMEM((CH, D), _f32),
            pltpu.SemaphoreType.DMA, pltpu.SemaphoreType.DMA,
            pltpu.SemaphoreType.DMA,
        ],
    )
    def k(P_h, Q_h, src_h, dst_h, gs_h, gd_h,
          isA, idA, isB, idB, rA1, rA2, rB1, rB2, semP, semQ, semW):
        cid = lax.axis_index("c")
        sid = lax.axis_index("s")
        wid = sid * 2 + cid

        def bump(dst_i, src_i, delta):
            for j in range(CH // 16):
                sl = pl.ds(j * 16, 16)
                dst_i[sl] = src_i[sl] + delta

        def body(kk, carry):
            g = wid + kk * 32

            @pl.when(g < NCHUNK)
            def _():
                eff = jnp.minimum(g * CH, E - CH)
                sl_e = pl.ds(eff, CH)
                pltpu.sync_copy(src_h.at[sl_e], isA)
                pltpu.sync_copy(dst_h.at[sl_e], idA)
                cpP = pltpu.async_copy(P_h.at[isA], rA1, semP)
                cpQ = pltpu.async_copy(Q_h.at[idA], rA2, semQ)
                bump(isB, isA, N)
                bump(idB, idA, N)
                cpP.wait()
                cpQ.wait()
                pltpu.async_copy(P_h.at[isB], rB1, semP)
                pltpu.async_copy(Q_h.at[idB], rB2, semQ)
                pltpu.async_copy(rA1, gs_h.at[0, sl_e], semW)
                pltpu.async_copy(rA2, gd_h.at[0, sl_e], semW)
                bump(isA, isA, 2 * N)
                bump(idA, idA, 2 * N)
                pltpu.make_async_copy(P_h.at[isB], rB1, semP).wait()
                pltpu.make_async_copy(Q_h.at[idB], rB2, semQ).wait()
                pltpu.make_async_copy(rA1, gs_h.at[0, sl_e], semW).wait()
                pltpu.make_async_copy(rA2, gd_h.at[0, sl_e], semW).wait()
                pltpu.async_copy(P_h.at[isA], rA1, semP)
                pltpu.async_copy(Q_h.at[idA], rA2, semQ)
                pltpu.async_copy(rB1, gs_h.at[1, sl_e], semW)
                pltpu.async_copy(rB2, gd_h.at[1, sl_e], semW)
                pltpu.make_async_copy(P_h.at[isA], rA1, semP).wait()
                pltpu.make_async_copy(Q_h.at[idA], rA2, semQ).wait()
                pltpu.make_async_copy(rB1, gs_h.at[1, sl_e], semW).wait()
                pltpu.make_async_copy(rB2, gd_h.at[1, sl_e], semW).wait()
                pltpu.sync_copy(rA1, gs_h.at[2, sl_e])
                pltpu.sync_copy(rA2, gd_h.at[2, sl_e])

            return carry

        lax.fori_loop(0, 32, body, 0)

    return k(Pf, Qf, src, dst)


def _sc_segsum(e0, e1, dst, zeros):
    @functools.partial(
        pl.kernel, mesh=_sc_mesh(),
        out_type=[jax.ShapeDtypeStruct((NM, N, HD), _f32),
                  jax.ShapeDtypeStruct((NM, N, HD), _f32)],
        scratch_types=[
            pltpu.VMEM((BIG,), jnp.int32),
            pltpu.VMEM((CH,), jnp.int32),
            pltpu.VMEM((CH,), jnp.int32),
            pltpu.VMEM((CH, HD), _f32),
            pltpu.VMEM((CH, HD), _f32),
            pltpu.VMEM((ZROWS // 8, HD), _f32),
            pltpu.VMEM_SHARED((SLAB_ROWS, HD), _f32),
            pltpu.SemaphoreType.DMA,
            pltpu.SemaphoreType.DMA,
        ],
    )
    def k(e0_h, e1_h, dst_h, z_h, a0_h, a1_h,
          idxbig, idxA, idxB, rows0, rows1, zbuf, slab, semA, semB):
        cid = lax.axis_index("c")
        sid = lax.axis_index("s")
        base_node = cid * HALF
        nrows = jnp.where(cid == 0, HALF, N - HALF)
        pltpu.sync_copy(z_h, zbuf)

        def eff_of(g):
            return jnp.minimum(g * CH, E - CH)

        def load_remap(g, idxv):
            raw = g * CH
            eff = eff_of(g)
            vfrom = raw - eff
            pltpu.sync_copy(dst_h.at[pl.ds(eff, CH)], idxv)
            for j in range(CH // 16):
                sl = pl.ds(j * 16, 16)
                li = idxv[sl] - base_node
                pos = lax.iota(jnp.int32, 16) + (j * 16)
                ok = (li >= 0) & (li < nrows) & (pos >= vfrom)
                idxv[sl] = jnp.where(ok, li, HALF)

        for m in range(NM):
            for e_h, agg_h in ((e0_h, a0_h), (e1_h, a1_h)):
                for z in range(8):
                    pltpu.sync_copy(zbuf, slab.at[pl.ds(sid * ZROWS + z * (ZROWS // 8),
                                                        ZROWS // 8)])
                plsc.subcore_barrier()

                pltpu.async_copy(e_h.at[m, pl.ds(eff_of(sid), CH)], rows0, semA)

                def pair(q, carry, m=m, e_h=e_h):
                    gA = sid + 32 * q
                    gB = gA + 16
                    gA2 = gA + 32

                    @pl.when(gB < NCHUNK)
                    def _():
                        pltpu.async_copy(e_h.at[m, pl.ds(eff_of(gB), CH)], rows1, semB)

                    load_remap(gA, idxA)
                    pltpu.make_async_copy(e_h.at[m, pl.ds(eff_of(gA), CH)], rows0, semA).wait()
                    pltpu.sync_copy(rows0, slab.at[idxA], add=True)

                    @pl.when(gA2 < NCHUNK)
                    def _():
                        pltpu.async_copy(e_h.at[m, pl.ds(eff_of(gA2), CH)], rows0, semA)

                    @pl.when(gB < NCHUNK)
                    def _():
                        load_remap(gB, idxB)
                        pltpu.make_async_copy(e_h.at[m, pl.ds(eff_of(gB), CH)], rows1, semB).wait()
                        pltpu.sync_copy(rows1, slab.at[idxB], add=True)

                    return carry

                lax.fori_loop(0, KMAX // 2, pair, 0)
                plsc.subcore_barrier()
                out_base = base_node + sid * ZROWS

                @pl.when((cid == 0) | (sid < 15))
                def _(m=m, agg_h=agg_h):
                    pltpu.sync_copy(slab.at[pl.ds(sid * ZROWS, ZROWS)],
                                    agg_h.at[m, pl.ds(out_base, ZROWS)])

                @pl.when((cid == 1) & (sid == 15))
                def _(m=m, agg_h=agg_h):
                    pltpu.sync_copy(slab.at[pl.ds(sid * ZROWS, TAIL_ROWS)],
                                    agg_h.at[m, pl.ds(out_base, TAIL_ROWS)])

                plsc.subcore_barrier()

    return k(e0, e1, dst, zeros)


def kernel(features, edge_index, edge_attr, params):
    p = params
    src = edge_index[0]
    dst = edge_index[1]
    x = features[0]

    beW1 = p["be_W1"]
    W1s, W1d, W1e = beW1[:, :, :D], beW1[:, :, D:2 * D], beW1[:, :, 2 * D:]
    bnW1 = p["bn_W1"]
    W1h = bnW1[:, :, :D]
    W1a0 = bnW1[:, :, D:D + HD]
    W1a1 = bnW1[:, :, D + HD:]

    zeros = jnp.zeros((ZROWS // 8, HD), _f32)

    h, P, Q = _enc_node(x, p["en_W1"], p["en_b1"], p["en_W2"], p["en_b2"],
                        p["en_g"], p["en_be"], W1s[:, 0], W1d[:, 0])
    e0, e1 = _enc_edge(edge_attr, p["ee_W1"], p["ee_b1"], p["ee_W2"], p["ee_b2"],
                       p["ee_g"], p["ee_be"])

    for b in range(NB):
        gs, gd = _sc_gather(P.reshape(NM * N, D), Q.reshape(NM * N, D), src, dst)
        e0, e1 = _edge_update(gs, gd, e0, e1, W1e[:, b], p["be_b1"][:, b], p["be_W2"][:, b],
                              p["be_b2"][:, b], p["be_g"][:, b], p["be_be"][:, b])
        agg0, agg1 = _sc_segsum(e0, e1, dst, zeros)
        nargs = (h, agg0, agg1, W1h[:, b], W1a0[:, b], W1a1[:, b], p["bn_b1"][:, b],
                 p["bn_W2"][:, b], p["bn_b2"][:, b], p["bn_g"][:, b], p["bn_be"][:, b])
        if b < NB - 1:
            h, P, Q = _node_update_proj(*nargs, W1s[:, b + 1], W1d[:, b + 1])
        else:
            h = _node_update(*nargs)

    sw = p["step_w"]
    return _decoder(h, p["de_W1"], p["de_b1"],
                    p["de_W2"] * sw[:, None, None], p["de_b2"] * sw[:, None])


# speedup vs baseline: 1.6536x; 1.0057x over previous
import functools

import jax
import jax.numpy as jnp
from jax import lax
from jax.experimental import pallas as pl
from jax.experimental.pallas import tpu as pltpu
from jax.experimental.pallas import tpu_sc as plsc

N = 16200
E = N * 8
F = 42
D = 128
HD = D // 2
DEC = 64
NB = 3
NM = 3

TRN = 1800
NRT = N // TRN
TRE = 1600
ERT = E // TRE

CH = 128
NCHUNK = (E + CH - 1) // CH
BIG = 1024
SUBE = 8192
NBIG = SUBE // BIG
HALF = 8192
SLAB_ROWS = HALF + 8
ZROWS = 512
TAIL_ROWS = N - HALF - 15 * ZROWS
KMAX = (NCHUNK + 15) // 16

_f32 = jnp.float32


def _layer_norm(x, g, b):
    m = jnp.mean(x, axis=-1, keepdims=True)
    v = jnp.mean((x - m) ** 2, axis=-1, keepdims=True)
    return (x - m) * lax.rsqrt(v + 1e-5) * g + b


def _bias3(v):
    return v.reshape(NM, 1, v.shape[-1])


def _wspec(shape):
    return pl.BlockSpec((1,) + shape, lambda m, r: (m, 0, 0))


def _espec_halves():
    sp = pl.BlockSpec((1, TRE, HD), lambda m, r: (m, r, 0))
    sh = [jax.ShapeDtypeStruct((NM, E, HD), _f32)] * 2
    return [sp, sp], sh


def _enc_node(x, W1, b1, W2, b2, g, be, Ws, Wd):
    def body(x_r, W1_r, b1_r, W2_r, b2_r, g_r, be_r, Ws_r, Wd_r, o_r, p_r, q_r):
        h1 = jax.nn.silu(jnp.dot(x_r[0], W1_r[0], preferred_element_type=_f32) + b1_r[0])
        u = jnp.dot(h1, W2_r[0], preferred_element_type=_f32) + b2_r[0]
        hb = _layer_norm(u, g_r[0], be_r[0])
        o_r[0] = hb
        p_r[0] = jnp.dot(hb, Ws_r[0], preferred_element_type=_f32)
        q_r[0] = jnp.dot(hb, Wd_r[0], preferred_element_type=_f32)

    spec = pl.BlockSpec((1, TRN, D), lambda m, r: (m, r, 0))
    return pl.pallas_call(
        body,
        grid=(NM, NRT),
        in_specs=[
            pl.BlockSpec((1, TRN, F), lambda m, r: (m, r, 0)),
            _wspec((F, D)), _wspec((1, D)), _wspec((D, D)),
            _wspec((1, D)), _wspec((1, D)), _wspec((1, D)),
            _wspec((D, D)), _wspec((D, D)),
        ],
        out_specs=[spec, spec, spec],
        out_shape=[jax.ShapeDtypeStruct((NM, N, D), _f32)] * 3,
    )(x, W1, _bias3(b1), W2, _bias3(b2), _bias3(g), _bias3(be), Ws, Wd)


def _enc_edge(ea, W1, b1, W2, b2, g, be):
    def body(x_r, W1_r, b1_r, W2_r, b2_r, g_r, be_r, o0_r, o1_r):
        h1 = jax.nn.silu(jnp.dot(x_r[...], W1_r[0], preferred_element_type=_f32) + b1_r[0])
        u = jnp.dot(h1, W2_r[0], preferred_element_type=_f32) + b2_r[0]
        res = _layer_norm(u, g_r[0], be_r[0])
        o0_r[0] = res[:, :HD]
        o1_r[0] = res[:, HD:]

    out_specs, out_shape = _espec_halves()
    return pl.pallas_call(
        body,
        grid=(NM, ERT),
        in_specs=[
            pl.BlockSpec((TRE, 4), lambda m, r: (r, 0)),
            _wspec((4, D)), _wspec((1, D)), _wspec((D, D)),
            _wspec((1, D)), _wspec((1, D)), _wspec((1, D)),
        ],
        out_specs=out_specs,
        out_shape=out_shape,
    )(ea, W1, _bias3(b1), W2, _bias3(b2), _bias3(g), _bias3(be))


def _node_body_core(h_r, a0_r, a1_r, W1h_r, Wa0_r, Wa1_r, b1_r, W2_r, b2_r, g_r, be_r):
    hb = h_r[0]
    t = jax.nn.silu(jnp.dot(hb, W1h_r[0], preferred_element_type=_f32)
                    + jnp.dot(a0_r[0], Wa0_r[0], preferred_element_type=_f32)
                    + jnp.dot(a1_r[0], Wa1_r[0], preferred_element_type=_f32)
                    + b1_r[0])
    u = jnp.dot(t, W2_r[0], preferred_element_type=_f32) + b2_r[0]
    return hb + _layer_norm(u, g_r[0], be_r[0])


def _node_update_proj(h, agg0, agg1, W1h, W1a0, W1a1, b1, W2, b2, g, be, Ws, Wd):
    def body(h_r, a0_r, a1_r, W1h_r, Wa0_r, Wa1_r, b1_r, W2_r, b2_r, g_r, be_r,
             Ws_r, Wd_r, o_r, p_r, q_r):
        hn = _node_body_core(h_r, a0_r, a1_r, W1h_r, Wa0_r, Wa1_r, b1_r, W2_r,
                             b2_r, g_r, be_r)
        o_r[0] = hn
        p_r[0] = jnp.dot(hn, Ws_r[0], preferred_element_type=_f32)
        q_r[0] = jnp.dot(hn, Wd_r[0], preferred_element_type=_f32)

    spec = pl.BlockSpec((1, TRN, D), lambda m, r: (m, r, 0))
    aspec = pl.BlockSpec((1, TRN, HD), lambda m, r: (m, r, 0))
    return pl.pallas_call(
        body,
        grid=(NM, NRT),
        in_specs=[spec, aspec, aspec, _wspec((D, D)), _wspec((HD, D)), _wspec((HD, D)),
                  _wspec((1, D)), _wspec((D, D)), _wspec((1, D)), _wspec((1, D)),
                  _wspec((1, D)), _wspec((D, D)), _wspec((D, D))],
        out_specs=[spec, spec, spec],
        out_shape=[jax.ShapeDtypeStruct((NM, N, D), _f32)] * 3,
    )(h, agg0, agg1, W1h, W1a0, W1a1, _bias3(b1), W2, _bias3(b2), _bias3(g),
      _bias3(be), Ws, Wd)


def _edge_update(gs, gd, e0, e1, W1e, b1, W2, b2, g, be):
    def body(gs_r, gd_r, e0_r, e1_r, W1e_r, b1_r, W2_r, b2_r, g_r, be_r, o0_r, o1_r):
        eb = jnp.concatenate([e0_r[0], e1_r[0]], axis=-1)
        t = jax.nn.silu(gs_r[0] + gd_r[0]
                        + jnp.dot(eb, W1e_r[0], preferred_element_type=_f32) + b1_r[0])
        u = jnp.dot(t, W2_r[0], preferred_element_type=_f32) + b2_r[0]
        res = eb + _layer_norm(u, g_r[0], be_r[0])
        o0_r[0] = res[:, :HD]
        o1_r[0] = res[:, HD:]

    spec = pl.BlockSpec((1, TRE, D), lambda m, r: (m, r, 0))
    hspec = pl.BlockSpec((1, TRE, HD), lambda m, r: (m, r, 0))
    out_specs, out_shape = _espec_halves()
    return pl.pallas_call(
        body,
        grid=(NM, ERT),
        in_specs=[spec, spec, hspec, hspec, _wspec((D, D)), _wspec((1, D)),
                  _wspec((D, D)), _wspec((1, D)), _wspec((1, D)), _wspec((1, D))],
        out_specs=out_specs,
        out_shape=out_shape,
    )(gs, gd, e0, e1, W1e, _bias3(b1), W2, _bias3(b2), _bias3(g), _bias3(be))


def _node_update(h, agg0, agg1, W1h, W1a0, W1a1, b1, W2, b2, g, be):
    def body(h_r, a0_r, a1_r, W1h_r, Wa0_r, Wa1_r, b1_r, W2_r, b2_r, g_r, be_r, o_r):
        o_r[0] = _node_body_core(h_r, a0_r, a1_r, W1h_r, Wa0_r, Wa1_r, b1_r,
                                 W2_r, b2_r, g_r, be_r)

    spec = pl.BlockSpec((1, TRN, D), lambda m, r: (m, r, 0))
    aspec = pl.BlockSpec((1, TRN, HD), lambda m, r: (m, r, 0))
    return pl.pallas_call(
        body,
        grid=(NM, NRT),
        in_specs=[spec, aspec, aspec, _wspec((D, D)), _wspec((HD, D)), _wspec((HD, D)),
                  _wspec((1, D)), _wspec((D, D)), _wspec((1, D)), _wspec((1, D)), _wspec((1, D))],
        out_specs=spec,
        out_shape=jax.ShapeDtypeStruct((NM, N, D), _f32),
    )(h, agg0, agg1, W1h, W1a0, W1a1, _bias3(b1), W2, _bias3(b2), _bias3(g), _bias3(be))


def _decoder(h, W1, b1, W2, b2):
    def body(h_r, W1_r, b1_r, W2_r, b2_r, o_r):
        m = pl.program_id(1)
        t = jax.nn.silu(jnp.dot(h_r[0], W1_r[0], preferred_element_type=_f32) + b1_r[0])
        z = jnp.dot(t, W2_r[0], preferred_element_type=_f32) + b2_r[0]

        @pl.when(m == 0)
        def _():
            o_r[...] = z

        @pl.when(m != 0)
        def _():
            o_r[...] = o_r[...] + z

    def wspec(shape):
        return pl.BlockSpec((1,) + shape, lambda r, m: (m, 0, 0))

    return pl.pallas_call(
        body,
        grid=(NRT, NM),
        in_specs=[
            pl.BlockSpec((1, TRN, D), lambda r, m: (m, r, 0)),
            wspec((D, DEC)), wspec((1, DEC)), wspec((DEC, F)), wspec((1, F)),
        ],
        out_specs=pl.BlockSpec((TRN, F), lambda r, m: (r, 0)),
        out_shape=jax.ShapeDtypeStruct((N, F), _f32),
    )(h, W1, _bias3(b1), W2, _bias3(b2))


def _sc_mesh():
    return plsc.VectorSubcoreMesh(core_axis_name="c", subcore_axis_name="s")


def _sc_gather(Pf, Qf, src, dst):
    @functools.partial(
        pl.kernel, mesh=_sc_mesh(),
        out_type=[jax.ShapeDtypeStruct((NM, E, D), _f32),
                  jax.ShapeDtypeStruct((NM, E, D), _f32)],
        scratch_types=[
            pltpu.VMEM((CH,), jnp.int32), pltpu.VMEM((CH,), jnp.int32),
            pltpu.VMEM((CH,), jnp.int32), pltpu.VMEM((CH,), jnp.int32),
            pltpu.VMEM((CH, D), _f32), pltpu.VMEM((CH, D), _f32),
            pltpu.VMEM((CH, D), _f32), pltpu.VMEM((CH, D), _f32),
            pltpu.SemaphoreType.DMA, pltpu.SemaphoreType.DMA,
            pltpu.SemaphoreType.DMA,
        ],
    )
    def k(P_h, Q_h, src_h, dst_h, gs_h, gd_h,
          isA, idA, isB, idB, rA1, rA2, rB1, rB2, semP, semQ, semW):
        cid = lax.axis_index("c")
        sid = lax.axis_index("s")
        wid = sid * 2 + cid

        def bump(dst_i, src_i, delta):
            for j in range(CH // 16):
                sl = pl.ds(j * 16, 16)
                dst_i[sl] = src_i[sl] + delta

        def body(kk, carry):
            g = wid + kk * 32

            @pl.when(g < NCHUNK)
            def _():
                eff = jnp.minimum(g * CH, E - CH)
                sl_e = pl.ds(eff, CH)
                pltpu.sync_copy(src_h.at[sl_e], isA)
                pltpu.sync_copy(dst_h.at[sl_e], idA)
                cpP = pltpu.async_copy(P_h.at[isA], rA1, semP)
                cpQ = pltpu.async_copy(Q_h.at[idA], rA2, semQ)
                bump(isB, isA, N)
                bump(idB, idA, N)
                cpP.wait()
                cpQ.wait()
                pltpu.async_copy(P_h.at[isB], rB1, semP)
                pltpu.async_copy(Q_h.at[idB], rB2, semQ)
                pltpu.async_copy(rA1, gs_h.at[0, sl_e], semW)
                pltpu.async_copy(rA2, gd_h.at[0, sl_e], semW)
                bump(isA, isA, 2 * N)
                bump(idA, idA, 2 * N)
                pltpu.make_async_copy(P_h.at[isB], rB1, semP).wait()
                pltpu.make_async_copy(Q_h.at[idB], rB2, semQ).wait()
                pltpu.make_async_copy(rA1, gs_h.at[0, sl_e], semW).wait()
                pltpu.make_async_copy(rA2, gd_h.at[0, sl_e], semW).wait()
                pltpu.async_copy(P_h.at[isA], rA1, semP)
                pltpu.async_copy(Q_h.at[idA], rA2, semQ)
                pltpu.async_copy(rB1, gs_h.at[1, sl_e], semW)
                pltpu.async_copy(rB2, gd_h.at[1, sl_e], semW)
                pltpu.make_async_copy(P_h.at[isA], rA1, semP).wait()
                pltpu.make_async_copy(Q_h.at[idA], rA2, semQ).wait()
                pltpu.make_async_copy(rB1, gs_h.at[1, sl_e], semW).wait()
                pltpu.make_async_copy(rB2, gd_h.at[1, sl_e], semW).wait()
                pltpu.sync_copy(rA1, gs_h.at[2, sl_e])
                pltpu.sync_copy(rA2, gd_h.at[2, sl_e])

            return carry

        lax.fori_loop(0, 32, body, 0)

    return k(Pf, Qf, src, dst)


def _sc_segsum(e0, e1, dst, zeros):
    @functools.partial(
        pl.kernel, mesh=_sc_mesh(),
        out_type=[jax.ShapeDtypeStruct((NM, N, HD), _f32),
                  jax.ShapeDtypeStruct((NM, N, HD), _f32)],
        scratch_types=[
            pltpu.VMEM((BIG,), jnp.int32),
            pltpu.VMEM((CH,), jnp.int32),
            pltpu.VMEM((CH,), jnp.int32),
            pltpu.VMEM((CH, HD), _f32),
            pltpu.VMEM((CH, HD), _f32),
            pltpu.VMEM((ZROWS // 8, HD), _f32),
            pltpu.VMEM_SHARED((SLAB_ROWS, HD), _f32),
            pltpu.SemaphoreType.DMA,
            pltpu.SemaphoreType.DMA,
            pltpu.SemaphoreType.DMA,
            pltpu.SemaphoreType.DMA,
        ],
    )
    def k(e0_h, e1_h, dst_h, z_h, a0_h, a1_h,
          idxbig, idxA, idxB, rows0, rows1, zbuf, slab, semA, semB, semIA, semIB):
        cid = lax.axis_index("c")
        sid = lax.axis_index("s")
        base_node = cid * HALF
        nrows = jnp.where(cid == 0, HALF, N - HALF)
        pltpu.sync_copy(z_h, zbuf)

        def eff_of(g):
            return jnp.minimum(g * CH, E - CH)

        def start_idx(g, idxv, sem):
            pltpu.async_copy(dst_h.at[pl.ds(eff_of(g), CH)], idxv, sem)

        def wait_remap(g, idxv, sem):
            raw = g * CH
            eff = eff_of(g)
            vfrom = raw - eff
            pltpu.make_async_copy(dst_h.at[pl.ds(eff, CH)], idxv, sem).wait()
            for j in range(CH // 16):
                sl = pl.ds(j * 16, 16)
                li = idxv[sl] - base_node
                pos = lax.iota(jnp.int32, 16) + (j * 16)
                ok = (li >= 0) & (li < nrows) & (pos >= vfrom)
                idxv[sl] = jnp.where(ok, li, HALF)

        for m in range(NM):
            for e_h, agg_h in ((e0_h, a0_h), (e1_h, a1_h)):
                for z in range(8):
                    pltpu.sync_copy(zbuf, slab.at[pl.ds(sid * ZROWS + z * (ZROWS // 8),
                                                        ZROWS // 8)])
                plsc.subcore_barrier()

                pltpu.async_copy(e_h.at[m, pl.ds(eff_of(sid), CH)], rows0, semA)
                start_idx(sid, idxA, semIA)

                def pair(q, carry, m=m, e_h=e_h):
                    gA = sid + 32 * q
                    gB = gA + 16
                    gA2 = gA + 32

                    @pl.when(gB < NCHUNK)
                    def _():
                        pltpu.async_copy(e_h.at[m, pl.ds(eff_of(gB), CH)], rows1, semB)
                        start_idx(gB, idxB, semIB)

                    wait_remap(gA, idxA, semIA)
                    pltpu.make_async_copy(e_h.at[m, pl.ds(eff_of(gA), CH)], rows0, semA).wait()
                    pltpu.sync_copy(rows0, slab.at[idxA], add=True)

                    @pl.when(gA2 < NCHUNK)
                    def _():
                        pltpu.async_copy(e_h.at[m, pl.ds(eff_of(gA2), CH)], rows0, semA)
                        start_idx(gA2, idxA, semIA)

                    @pl.when(gB < NCHUNK)
                    def _():
                        wait_remap(gB, idxB, semIB)
                        pltpu.make_async_copy(e_h.at[m, pl.ds(eff_of(gB), CH)], rows1, semB).wait()
                        pltpu.sync_copy(rows1, slab.at[idxB], add=True)

                    return carry

                lax.fori_loop(0, KMAX // 2, pair, 0)
                plsc.subcore_barrier()
                out_base = base_node + sid * ZROWS

                @pl.when((cid == 0) | (sid < 15))
                def _(m=m, agg_h=agg_h):
                    pltpu.sync_copy(slab.at[pl.ds(sid * ZROWS, ZROWS)],
                                    agg_h.at[m, pl.ds(out_base, ZROWS)])

                @pl.when((cid == 1) & (sid == 15))
                def _(m=m, agg_h=agg_h):
                    pltpu.sync_copy(slab.at[pl.ds(sid * ZROWS, TAIL_ROWS)],
                                    agg_h.at[m, pl.ds(out_base, TAIL_ROWS)])

                plsc.subcore_barrier()

    return k(e0, e1, dst, zeros)


def kernel(features, edge_index, edge_attr, params):
    p = params
    src = edge_index[0]
    dst = edge_index[1]
    x = features[0]

    beW1 = p["be_W1"]
    W1s, W1d, W1e = beW1[:, :, :D], beW1[:, :, D:2 * D], beW1[:, :, 2 * D:]
    bnW1 = p["bn_W1"]
    W1h = bnW1[:, :, :D]
    W1a0 = bnW1[:, :, D:D + HD]
    W1a1 = bnW1[:, :, D + HD:]

    zeros = jnp.zeros((ZROWS // 8, HD), _f32)

    h, P, Q = _enc_node(x, p["en_W1"], p["en_b1"], p["en_W2"], p["en_b2"],
                        p["en_g"], p["en_be"], W1s[:, 0], W1d[:, 0])
    e0, e1 = _enc_edge(edge_attr, p["ee_W1"], p["ee_b1"], p["ee_W2"], p["ee_b2"],
                       p["ee_g"], p["ee_be"])

    for b in range(NB):
        gs, gd = _sc_gather(P.reshape(NM * N, D), Q.reshape(NM * N, D), src, dst)
        e0, e1 = _edge_update(gs, gd, e0, e1, W1e[:, b], p["be_b1"][:, b], p["be_W2"][:, b],
                              p["be_b2"][:, b], p["be_g"][:, b], p["be_be"][:, b])
        agg0, agg1 = _sc_segsum(e0, e1, dst, zeros)
        nargs = (h, agg0, agg1, W1h[:, b], W1a0[:, b], W1a1[:, b], p["bn_b1"][:, b],
                 p["bn_W2"][:, b], p["bn_b2"][:, b], p["bn_g"][:, b], p["bn_be"][:, b])
        if b < NB - 1:
            h, P, Q = _node_update_proj(*nargs, W1s[:, b + 1], W1d[:, b + 1])
        else:
            h = _node_update(*nargs)

    sw = p["step_w"]
    return _decoder(h, p["de_W1"], p["de_b1"],
                    p["de_W2"] * sw[:, None, None], p["de_b2"] * sw[:, None])
